# baseline (speedup 1.0000x reference)
import dataclasses
import functools

import jax
import jax.numpy as jnp
import numpy as np
from jax import lax
from jax.experimental import pallas as pl
from jax.experimental.pallas import tpu as pltpu
from jax.experimental.pallas import tpu_sc as plsc

VOCAB = 100000
D_MODEL = 768
MAX_SEQ = 2048
BATCH = 4

NUM_CORES = 2
NUM_SUBCORES = 16
NUM_WORKERS = NUM_CORES * NUM_SUBCORES
TOTAL = BATCH * MAX_SEQ
POS_PER_W = MAX_SEQ // NUM_WORKERS
B_PER_W = POS_PER_W * BATCH
CHUNK = 16
CHUNKS_PER_SEG = POS_PER_W // CHUNK
N_CHUNKS = B_PER_W // CHUNK
NGBUF = 4
NOBUF = 2
LANES = 16
NLG = D_MODEL // LANES


def _positional_encoding() -> np.ndarray:
    pos = np.arange(MAX_SEQ, dtype=np.float32)[:, None]
    dim = np.arange(0, D_MODEL, 2, dtype=np.float32)
    angle = pos / np.power(10000.0, dim / D_MODEL, dtype=np.float32)
    pe = np.zeros((MAX_SEQ, D_MODEL), dtype=np.float32)
    pe[:, 0::2] = np.sin(angle)
    pe[:, 1::2] = np.cos(angle)
    return pe


_PE = _positional_encoding()


def _compiler_params():
    cp = pltpu.CompilerParams()
    if "needs_layout_passes" in pltpu.CompilerParams.__dataclass_fields__:
        cp = dataclasses.replace(cp, needs_layout_passes=False)
    return cp


def _sc_embed(table, ids_flat, pe):
    mesh = plsc.VectorSubcoreMesh(core_axis_name="c", subcore_axis_name="s")

    @functools.partial(
        pl.kernel,
        out_type=jax.ShapeDtypeStruct((TOTAL, D_MODEL), jnp.float32),
        mesh=mesh,
        compiler_params=_compiler_params(),
        scratch_types=[
            pltpu.VMEM((B_PER_W,), jnp.int32),
            pltpu.VMEM((POS_PER_W * D_MODEL,), jnp.float32),
            [pltpu.VMEM((CHUNK, D_MODEL), jnp.float32) for _ in range(NGBUF)],
            [pltpu.VMEM((CHUNK, D_MODEL), jnp.float32) for _ in range(NOBUF)],
            pltpu.SemaphoreType.DMA,
            [pltpu.SemaphoreType.DMA for _ in range(NGBUF)],
            [pltpu.SemaphoreType.DMA for _ in range(NOBUF)],
        ],
    )
    def k(table_hbm, idx_hbm, pe_hbm, out_hbm, idx_v, pe_v,
          gbufs, obufs, pe_sem, gsems, osems):
        wid = lax.axis_index("s") * NUM_CORES + lax.axis_index("c")
        pos_base = wid * POS_PER_W

        pe_cp = pltpu.make_async_copy(
            pe_hbm.at[pl.ds(pos_base * D_MODEL, POS_PER_W * D_MODEL)], pe_v, pe_sem
        )
        pe_cp.start()
        for b in range(BATCH):
            pltpu.sync_copy(
                idx_hbm.at[b, pl.ds(pos_base, POS_PER_W)],
                idx_v.at[pl.ds(b * POS_PER_W, POS_PER_W)],
            )

        def gather_copy(c, b):
            return pltpu.make_async_copy(
                table_hbm.at[idx_v.at[pl.ds(c * CHUNK, CHUNK)]], gbufs[b], gsems[b]
            )

        def out_copy(seg_off, h, b):
            return pltpu.make_async_copy(
                obufs[b],
                out_hbm.at[pl.ds(seg_off + pos_base + h * CHUNK, CHUNK), :],
                osems[b],
            )

        def drain_out(b):
            pltpu.make_async_copy(
                obufs[b], out_hbm.at[pl.ds(pos_base, CHUNK), :], osems[b]
            ).wait()

        for c in range(NGBUF - 1):
            gather_copy(c, c).start()
        pe_cp.wait()

        @pl.loop(0, N_CHUNKS, step=4)
        def _(c0):
            seg_off = c0 * (MAX_SEQ // CHUNKS_PER_SEG)
            for i in range(4):
                c = c0 + i
                bg = i % NGBUF
                bo = i % NOBUF
                pe_row0 = i * CHUNK
                gather_copy(c, bg).wait()

                @pl.when(c >= NOBUF)
                def _():
                    drain_out(bo)

                gbuf, obuf = gbufs[bg], obufs[bo]

                @plsc.parallel_loop(0, CHUNK)
                def _(r):
                    base = (pe_row0 + r) * D_MODEL
                    for j in range(NLG):
                        sl = pl.ds(j * LANES, LANES)
                        obuf[r, sl] = gbuf[r, sl] + pe_v[pl.ds(base + j * LANES, LANES)]

                out_copy(seg_off, i, bo).start()

                @pl.when(c + NGBUF - 1 < N_CHUNKS)
                def _():
                    gather_copy(c + NGBUF - 1, (i + NGBUF - 1) % NGBUF).start()

        for c in range(N_CHUNKS - NOBUF, N_CHUNKS):
            drain_out(c % NOBUF)

    return k(table, ids_flat, pe)


def kernel(input_ids, emb_table):
    bs, seq = input_ids.shape
    ids2d = input_ids.astype(jnp.int32)
    out = _sc_embed(emb_table, ids2d, jnp.asarray(_PE.reshape(-1)))
    return out.reshape(bs, seq, D_MODEL)


# speedup vs baseline: 1.1943x; 1.1943x over previous
import dataclasses
import functools

import jax
import jax.numpy as jnp
import numpy as np
from jax import lax
from jax.experimental import pallas as pl
from jax.experimental.pallas import tpu as pltpu
from jax.experimental.pallas import tpu_sc as plsc

VOCAB = 100000
D_MODEL = 768
MAX_SEQ = 2048
BATCH = 4

NUM_CORES = 2
NUM_SUBCORES = 16
NUM_WORKERS = NUM_CORES * NUM_SUBCORES
TOTAL = BATCH * MAX_SEQ
POS_PER_W = MAX_SEQ // NUM_WORKERS
B_PER_W = POS_PER_W * BATCH
CHUNK = 16
CHUNKS_PER_SEG = POS_PER_W // CHUNK
N_CHUNKS = B_PER_W // CHUNK
NGBUF = 4
NOBUF = 2
LANES = 16
NLG = D_MODEL // LANES


def _positional_encoding() -> np.ndarray:
    pos = np.arange(MAX_SEQ, dtype=np.float32)[:, None]
    dim = np.arange(0, D_MODEL, 2, dtype=np.float32)
    angle = pos / np.power(10000.0, dim / D_MODEL, dtype=np.float32)
    pe = np.zeros((MAX_SEQ, D_MODEL), dtype=np.float32)
    pe[:, 0::2] = np.sin(angle)
    pe[:, 1::2] = np.cos(angle)
    return pe


_PE = _positional_encoding()


def _compiler_params():
    cp = pltpu.CompilerParams()
    if "needs_layout_passes" in pltpu.CompilerParams.__dataclass_fields__:
        cp = dataclasses.replace(cp, needs_layout_passes=False)
    return cp


def _sc_embed(table, ids_flat, pe):
    mesh = plsc.VectorSubcoreMesh(core_axis_name="c", subcore_axis_name="s")

    @functools.partial(
        pl.kernel,
        out_type=jax.ShapeDtypeStruct((TOTAL, D_MODEL), jnp.float32),
        mesh=mesh,
        compiler_params=_compiler_params(),
        scratch_types=[
            pltpu.VMEM((B_PER_W,), jnp.int32),
            pltpu.VMEM((POS_PER_W, D_MODEL), jnp.float32),
            [pltpu.VMEM((CHUNK, D_MODEL), jnp.float32) for _ in range(NGBUF)],
            [pltpu.VMEM((CHUNK, D_MODEL), jnp.float32) for _ in range(NOBUF)],
            pltpu.SemaphoreType.DMA,
            [pltpu.SemaphoreType.DMA for _ in range(NGBUF)],
            [pltpu.SemaphoreType.DMA for _ in range(NOBUF)],
        ],
    )
    def k(table_hbm, idx_hbm, pe_hbm, out_hbm, idx_v, pe_v,
          gbufs, obufs, pe_sem, gsems, osems):
        wid = lax.axis_index("s") * NUM_CORES + lax.axis_index("c")
        pos_base = wid * POS_PER_W

        pe_cp = pltpu.make_async_copy(
            pe_hbm.at[pl.ds(pos_base, POS_PER_W), :], pe_v, pe_sem
        )
        pe_cp.start()
        for b in range(BATCH):
            pltpu.sync_copy(
                idx_hbm.at[b, pl.ds(pos_base, POS_PER_W)],
                idx_v.at[pl.ds(b * POS_PER_W, POS_PER_W)],
            )

        def gather_copy(c, b):
            return pltpu.make_async_copy(
                table_hbm.at[idx_v.at[pl.ds(c * CHUNK, CHUNK)]], gbufs[b], gsems[b]
            )

        def out_copy(seg_off, h, b):
            return pltpu.make_async_copy(
                obufs[b],
                out_hbm.at[pl.ds(seg_off + pos_base + h * CHUNK, CHUNK), :],
                osems[b],
            )

        def drain_out(b):
            pltpu.make_async_copy(
                obufs[b], out_hbm.at[pl.ds(pos_base, CHUNK), :], osems[b]
            ).wait()

        for c in range(NGBUF - 1):
            gather_copy(c, c).start()
        pe_cp.wait()

        @pl.loop(0, N_CHUNKS, step=4)
        def _(c0):
            seg_off = c0 * (MAX_SEQ // CHUNKS_PER_SEG)
            for i in range(4):
                c = c0 + i
                bg = i % NGBUF
                bo = i % NOBUF
                pe_row0 = i * CHUNK
                gather_copy(c, bg).wait()

                @pl.when(c >= NOBUF)
                def _():
                    drain_out(bo)

                gbuf, obuf = gbufs[bg], obufs[bo]

                @plsc.parallel_loop(0, CHUNK)
                def _(r):
                    for j in range(NLG):
                        sl = pl.ds(j * LANES, LANES)
                        obuf[r, sl] = gbuf[r, sl] + pe_v[pe_row0 + r, sl]

                out_copy(seg_off, i, bo).start()

                @pl.when(c + NGBUF - 1 < N_CHUNKS)
                def _():
                    gather_copy(c + NGBUF - 1, (i + NGBUF - 1) % NGBUF).start()

        for c in range(N_CHUNKS - NOBUF, N_CHUNKS):
            drain_out(c % NOBUF)

    return k(table, ids_flat, pe)


def kernel(input_ids, emb_table):
    bs, seq = input_ids.shape
    ids2d = input_ids.astype(jnp.int32)
    out = _sc_embed(emb_table, ids2d, jnp.asarray(_PE))
    return out.reshape(bs, seq, D_MODEL)
